# baseline (speedup 1.0000x reference)
import functools

import jax
import jax.numpy as jnp
from jax import lax
from jax.experimental import pallas as pl
from jax.experimental.pallas import tpu as pltpu
from jax.experimental.pallas import tpu_sc as plsc

_LANES = 16
_CH = 128


@jax.jit
def kernel(node, time, embedding):
    B = node.shape[0]
    N, D = embedding.shape
    pack = 128 // D
    info = plsc.get_sparse_core_info()
    nw = info.num_cores * info.num_subcores
    b_per_w = B // nw
    n_chunks = b_per_w // _CH

    mesh = plsc.VectorSubcoreMesh(core_axis_name="c", subcore_axis_name="s")

    @functools.partial(
        pl.kernel,
        mesh=mesh,
        compiler_params=pltpu.CompilerParams(needs_layout_passes=False),
        out_type=jax.ShapeDtypeStruct((nw, b_per_w), jnp.float32),
        scratch_types=[
            pltpu.VMEM((n_chunks, _CH), jnp.int32),
            pltpu.VMEM((n_chunks, _CH), jnp.int32),
            pltpu.VMEM((n_chunks, _CH), jnp.int32),
            pltpu.VMEM((n_chunks, _CH), jnp.int32),
            pltpu.VMEM((2, _CH, 128), jnp.float32),
            pltpu.VMEM((2, _CH, 128), jnp.float32),
            pltpu.VMEM((b_per_w,), jnp.float32),
            pltpu.VMEM((_LANES * _LANES,), jnp.float32),
            pltpu.SemaphoreType.DMA,
            pltpu.SemaphoreType.DMA,
        ],
    )
    def sc_kernel(node_hbm, time_hbm, emb_hbm, out_hbm,
                  idx_n, idx_t, q_n, q_t, rows_n, rows_t, out_v, pbuf,
                  sem_a, sem_b):
        c = lax.axis_index("c")
        s = lax.axis_index("s")
        wid = s * info.num_cores + c

        pltpu.sync_copy(node_hbm.at[wid], idx_n)
        pltpu.sync_copy(time_hbm.at[wid], idx_t)

        for j in range(n_chunks):
            for k in range(_CH // _LANES):
                ks = pl.ds(k * _LANES, _LANES)
                q_n[j, ks] = idx_n[j, ks] >> 2
                q_t[j, ks] = idx_t[j, ks] >> 2

        sems = (sem_a, sem_b)

        def fire(j):
            buf = j % 2
            return (
                pltpu.async_copy(emb_hbm.at[q_n.at[j]], rows_n.at[buf],
                                 sems[buf]),
                pltpu.async_copy(emb_hbm.at[q_t.at[j]], rows_t.at[buf],
                                 sems[buf]),
            )

        sidx = lax.iota(jnp.int32, _LANES) * _LANES

        def chunk_compute(j):
            rn = rows_n.at[j % 2]
            rt = rows_t.at[j % 2]

            def grp(g, carry):
                row0 = g * _LANES
                gs = pl.ds(row0, _LANES)
                ovn = (idx_n[j, gs] & (pack - 1)) * D
                ovt = (idx_t[j, gs] & (pack - 1)) * D
                for r in range(_LANES):
                    row = row0 + r
                    on = ovn[r]
                    ot = ovt[r]
                    pv = jnp.zeros((_LANES,), jnp.float32)
                    for h in range(D // _LANES):
                        vn = rn[row, pl.ds(on + h * _LANES, _LANES)]
                        vt = rt[row, pl.ds(ot + h * _LANES, _LANES)]
                        pv = pv + vn * vt
                    plsc.store_scatter(pbuf, [sidx + r], pv)
                acc = jnp.zeros((_LANES,), jnp.float32)
                for k in range(_LANES):
                    acc = acc + pbuf[pl.ds(k * _LANES, _LANES)]
                out_v[pl.ds(j * _CH + row0, _LANES)] = acc
                return carry

            lax.fori_loop(0, _CH // _LANES, grp, 0)

        pending = {0: fire(0)}
        for j in range(n_chunks):
            if j + 1 < n_chunks:
                pending[j + 1] = fire(j + 1)
            for cp in pending.pop(j):
                cp.wait()
            chunk_compute(j)

        pltpu.sync_copy(out_v, out_hbm.at[wid])

    node_r = node.astype(jnp.int32).reshape(nw, n_chunks, _CH)
    time_r = time.astype(jnp.int32).reshape(nw, n_chunks, _CH)
    emb_r = embedding.reshape(N // pack, 128)
    out = sc_kernel(node_r, time_r, emb_r)
    return out.reshape(B)


# speedup vs baseline: 1.0321x; 1.0321x over previous
import functools

import jax
import jax.numpy as jnp
from jax import lax
from jax.experimental import pallas as pl
from jax.experimental.pallas import tpu as pltpu
from jax.experimental.pallas import tpu_sc as plsc

_L = 16
_TILE = 128
_CHT = 8
_RING = 256


@jax.jit
def kernel(node, time, embedding):
    B = node.shape[0]
    N, D = embedding.shape
    embT = embedding.T
    n_tiles = N // _TILE + 1
    tail_n = N - (n_tiles - 1) * _TILE
    tail = jnp.pad(embT[:, N - tail_n:], ((0, 0), (0, _TILE - tail_n)))
    tail = tail.reshape(D // 8, 8, _TILE)

    info = plsc.get_sparse_core_info()
    nsc = info.num_cores
    nsub = info.num_subcores
    nw = nsc * nsub
    base_t, extra = divmod(n_tiles, nw)
    n_chunks = -(-(base_t + 1) // _CHT)
    stage_words = 2 * B * D
    sh_words = stage_words // nsub

    mesh = plsc.VectorSubcoreMesh(core_axis_name="c", subcore_axis_name="s")

    @functools.partial(
        pl.kernel,
        mesh=mesh,
        compiler_params=pltpu.CompilerParams(needs_layout_passes=False),
        out_type=jax.ShapeDtypeStruct((stage_words,), jnp.float32),
        scratch_types=[
            pltpu.VMEM((2048,), jnp.int32),
            pltpu.VMEM((2080,), jnp.int32),
            pltpu.VMEM((2080,), jnp.int32),
            pltpu.VMEM((2, _CHT, D // 8, 8, _TILE), jnp.float32),
            pltpu.VMEM((_RING, D), jnp.float32),
            pltpu.SMEM((1,), jnp.int32),
            pltpu.SemaphoreType.DMA,
            pltpu.SemaphoreType.DMA,
        ],
    )
    def sc_gather(node_hbm, time_hbm, embT_hbm, tail_hbm, s_hbm,
                  idxwin, hit_idx, hit_pos, slab, ring, mcnt,
                  sem_sw, sem_st):
        c = lax.axis_index("c")
        s = lax.axis_index("s")
        w = c * nsub + s
        lo_t = w * base_t + jnp.minimum(w, extra)
        my_t = base_t + jnp.where(w < extra, 1, 0)
        hi_t = lo_t + my_t
        lo_n = lo_t * _TILE
        hi_n = hi_t * _TILE

        lanes = lax.iota(jnp.int32, _L)
        W = 2048

        nh = 0
        for a, src in ((0, node_hbm), (1, time_hbm)):
            def piece(p, nh_c, a=a, src=src):
                pltpu.sync_copy(src.at[pl.ds(p * W, W)], idxwin)

                def scan(i, nh_i, a=a, p=p):
                    iv = idxwin[pl.ds(i * _L, _L)]
                    m = (iv >= lo_n) & (iv < hi_n)
                    cnt = plsc.all_reduce_population_count(m)[0]
                    pv = a * B + p * W + i * _L + lanes
                    plsc.store_compressed(
                        hit_idx.at[pl.ds(nh_i, _L)], iv, mask=m)
                    plsc.store_compressed(
                        hit_pos.at[pl.ds(nh_i, _L)], pv, mask=m)
                    return nh_i + cnt
                return lax.fori_loop(0, W // _L, scan, nh_c)
            nh = lax.fori_loop(0, B // W, piece, nh)
        hit_idx[pl.ds(nh, _L)] = jnp.full((_L,), -1, jnp.int32)
        mcnt[0] = 0

        def fire(ch):
            buf = ch & 1
            t0 = lo_t + ch * _CHT
            nt = jnp.clip(hi_t - t0, 0, _CHT)

            def body(j, carry):
                ti = j >> 2
                dt = j & 3
                tile = t0 + ti

                @pl.when(tile == n_tiles - 1)
                def _():
                    pltpu.async_copy(tail_hbm.at[dt], slab.at[buf, ti, dt],
                                     sem_sw)

                @pl.when(tile < n_tiles - 1)
                def _():
                    pltpu.async_copy(
                        embT_hbm.at[pl.ds(dt * 8, 8),
                                    pl.ds(pl.multiple_of(tile * _TILE, _TILE),
                                          _TILE)],
                        slab.at[buf, ti, dt], sem_sw)
                return carry
            lax.fori_loop(0, nt * 4, body, 0)

        def drain(ch):
            buf = ch & 1
            t0 = lo_t + ch * _CHT
            nt = jnp.clip(hi_t - t0, 0, _CHT)

            def body(j, carry):
                ti = j >> 2
                dt = j & 3
                pltpu.make_async_copy(
                    tail_hbm.at[dt], slab.at[buf, ti, dt], sem_sw).wait()
                return carry
            lax.fori_loop(0, nt * 4, body, 0)

        fdt = lanes >> 3
        fsv = lanes & 7

        def process(ch):
            buf = ch & 1
            t0 = lo_t + ch * _CHT
            clo = t0 * _TILE
            chi = jnp.minimum(t0 + _CHT, hi_t) * _TILE
            nv = (nh + _L - 1) >> 4

            def rescan(k, carry):
                hv = hit_idx[pl.ds(k * _L, _L)]
                pv = hit_pos[pl.ds(k * _L, _L)]
                m2 = (hv >= clo) & (hv < chi)
                im = jnp.where(m2, 1, 0)
                any_hit = plsc.all_reduce_population_count(m2)[0]

                @pl.when(any_hit > 0)
                def _():
                    for r in range(_L):
                        @pl.when(im[r] == 1)
                        def _():
                            idx = hv[r]
                            pos = pv[r]
                            ti = (idx >> 7) - t0
                            lane = idx & (_TILE - 1)
                            bufv = jnp.full((_L,), buf, jnp.int32)
                            tiv = jnp.full((_L,), ti, jnp.int32)
                            lv = jnp.full((_L,), lane, jnp.int32)
                            v0 = plsc.load_gather(
                                slab, [bufv, tiv, fdt, fsv, lv])
                            v1 = plsc.load_gather(
                                slab, [bufv, tiv, fdt + 2, fsv, lv])
                            m = mcnt[0]
                            slot = m & (_RING - 1)

                            @pl.when(m >= _RING)
                            def _():
                                pltpu.make_async_copy(
                                    tail_hbm.at[0, 0, pl.ds(0, D)],
                                    ring.at[slot], sem_st).wait()
                            ring[slot, pl.ds(0, _L)] = v0
                            ring[slot, pl.ds(_L, _L)] = v1
                            pltpu.async_copy(
                                ring.at[slot],
                                s_hbm.at[pl.ds(pos * D, D)], sem_st)
                            mcnt[0] = m + 1
                return carry
            lax.fori_loop(0, nv, rescan, 0)

        fire(0)

        def chunk_loop(ch, carry):
            @pl.when(ch + 1 < n_chunks)
            def _():
                fire(ch + 1)
            drain(ch)
            process(ch)
            return carry
        lax.fori_loop(0, n_chunks, chunk_loop, 0)

        mfin = jnp.minimum(mcnt[0], _RING)

        def fdrain(i, carry):
            pltpu.make_async_copy(
                tail_hbm.at[0, 0, pl.ds(0, D)], ring.at[0], sem_st).wait()
            return carry
        lax.fori_loop(0, mfin, fdrain, 0)


    node_i = node.astype(jnp.int32)
    time_i = time.astype(jnp.int32)
    sarr = sc_gather(node_i, time_i, embT, tail)

    rows = stage_words // _TILE
    half = rows // 2
    sr = sarr.reshape(rows, _TILE)
    blk = 1024
    grid = half // blk

    def dot_kernel(sn, st, o):
        p = sn[...] * st[...]
        seg = jax.lax.broadcasted_iota(jnp.int32, (_TILE, _TILE // D), 0) // D
        col = jax.lax.broadcasted_iota(jnp.int32, (_TILE, _TILE // D), 1)
        m = jnp.where(seg == col, 1.0, 0.0).astype(jnp.float32)
        o[...] = jax.lax.dot_general(
            p, m, (((1,), (0,)), ((), ())),
            preferred_element_type=jnp.float32)

    out4 = pl.pallas_call(
        dot_kernel,
        grid=(grid,),
        in_specs=[
            pl.BlockSpec((blk, _TILE), lambda i: (i, 0)),
            pl.BlockSpec((blk, _TILE), lambda i: (i + grid, 0)),
        ],
        out_specs=pl.BlockSpec((blk, _TILE // D), lambda i: (i, 0)),
        out_shape=jax.ShapeDtypeStruct((half, _TILE // D), jnp.float32),
    )(sr, sr)

    return out4.reshape(B)
